# baseline (device time: 47238 ns/iter reference)
import jax
import jax.numpy as jnp
from jax import lax
from jax.experimental import pallas as pl
from jax.experimental.pallas import tpu as pltpu

N_DEV = 4
CH = 256
NSUB = 4
SUB = CH // NSUB
HALF = N_DEV * CH
NHOP = 2 * (N_DEV - 1)


def _f(s):
    r = jnp.maximum(s, 0.0)
    return jnp.tanh(s) * s * s + r * r * r


def kernel(t):
    m, n = t.shape
    assert m == 2 * HALF

    def body(x_hbm, out_hbm, x_v, out_v, rbf, sbf, agb,
             ssem, rsem, insem, outsem):
        my = lax.axis_index("i")
        right = lax.rem(my + 1, N_DEV)
        left = lax.rem(my + 3, N_DEV)

        def rows(d, q, s=None):
            if s is None:
                return pl.ds(d * HALF + q * CH, CH)
            return pl.ds(d * HALF + q * CH + s * SUB, SUB)

        def agrows(q, s):
            return pl.ds(q * CH + s * SUB, SUB)

        def chunk_id(d, h):
            if h < N_DEV - 1:
                delta = -h if d == 0 else h
            else:
                ha = h - (N_DEV - 1)
                delta = 1 - ha if d == 0 else -1 + ha
            return lax.rem(my + delta + 2 * N_DEV, N_DEV)

        in_copies = {}
        for k, (df, dr) in enumerate(zip([0, -1, -2, 1], [0, 1, 2, -1])):
            for d, delta in ((0, df), (1, dr)):
                c = lax.rem(my + delta + N_DEV, N_DEV)
                cp = pltpu.make_async_copy(
                    x_hbm.at[rows(d, c)], x_v.at[rows(d, c)], insem.at[d, k]
                )
                cp.start()
                in_copies[(d, k)] = cp

        barrier_sem = pltpu.get_barrier_semaphore()
        for nbr in [left, right]:
            pl.semaphore_signal(
                barrier_sem, inc=1,
                device_id=(nbr,), device_id_type=pl.DeviceIdType.MESH,
            )

        for d in (0, 1):
            in_copies[(d, 0)].wait()
            c0 = chunk_id(d, 0)
            sbf[d, 0, :, :] = x_v[rows(d, c0), :].astype(jnp.bfloat16)

        pl.semaphore_wait(barrier_sem, 2)

        def rdma(src, dst, d, s, h, dev):
            return pltpu.make_async_remote_copy(
                src_ref=src, dst_ref=dst,
                send_sem=ssem.at[d, s, h], recv_sem=rsem.at[d, s, h],
                device_id=(dev,), device_id_type=pl.DeviceIdType.MESH,
            )

        dev_of = {0: right, 1: left}
        started = {}
        out_copies = []
        order = [(d, s) for s in range(NSUB) for d in (0, 1)]

        def copy_out(region):
            cp = pltpu.make_async_copy(
                out_v.at[region], out_hbm.at[region], outsem.at[len(out_copies)]
            )
            cp.start()
            out_copies.append(cp)

        for h in range(N_DEV - 1):
            for d, s in order:
                c = chunk_id(d, h)
                if s == 0 and h > 0:
                    in_copies[(d, h)].wait()
                sub = pl.ds(s * SUB, SUB)
                if h == 0:
                    pass
                else:
                    started[(d, s, h - 1)].wait_recv()
                    sbf[d, h, sub, :] = (
                        rbf[d, h - 1, sub, :].astype(jnp.float32)
                        + x_v[rows(d, c, s), :]
                    ).astype(jnp.bfloat16)
                r = rdma(sbf.at[d, h, sub], rbf.at[d, h, sub], d, s, h, dev_of[d])
                r.start()
                started[(d, s, h)] = r

        for d, s in order:
            started[(d, s, N_DEV - 2)].wait_recv()
            if s == 0:
                in_copies[(d, N_DEV - 1)].wait()
            c = chunk_id(d, N_DEV - 1)
            sub = pl.ds(s * SUB, SUB)
            y = _f(
                rbf[d, N_DEV - 2, sub, :].astype(jnp.float32)
                + x_v[rows(d, c, s), :]
            )
            out_v[rows(d, c, s), :] = y
            agb[d, agrows(c, s), :] = y.astype(jnp.bfloat16)
            r = rdma(agb.at[d, agrows(c, s)], agb.at[d, agrows(c, s)],
                     d, s, N_DEV - 1, dev_of[d])
            r.start()
            started[(d, s, N_DEV - 1)] = r
            copy_out(rows(d, c, s))

        for h in range(N_DEV, NHOP):
            for d, s in order:
                started[(d, s, h - 1)].wait_recv()
                c = chunk_id(d, h)
                r = rdma(agb.at[d, agrows(c, s)], agb.at[d, agrows(c, s)],
                         d, s, h, dev_of[d])
                r.start()
                started[(d, s, h)] = r
                out_v[rows(d, c, s), :] = agb[d, agrows(c, s), :].astype(
                    jnp.float32
                )
                copy_out(rows(d, c, s))

        for d, s in order:
            started[(d, s, NHOP - 1)].wait_recv()
            cf = chunk_id(d, NHOP)
            out_v[rows(d, cf, s), :] = agb[d, agrows(cf, s), :].astype(
                jnp.float32
            )
            copy_out(rows(d, cf, s))
        for cp in out_copies:
            cp.wait()
        for r in started.values():
            r.wait_send()

    return pl.pallas_call(
        body,
        out_shape=jax.ShapeDtypeStruct((m, n), jnp.float32),
        in_specs=[pl.BlockSpec(memory_space=pl.ANY)],
        out_specs=pl.BlockSpec(memory_space=pl.ANY),
        scratch_shapes=[
            pltpu.VMEM((m, n), jnp.float32),
            pltpu.VMEM((m, n), jnp.float32),
            pltpu.VMEM((2, N_DEV - 1, CH, n), jnp.bfloat16),
            pltpu.VMEM((2, N_DEV - 1, CH, n), jnp.bfloat16),
            pltpu.VMEM((2, HALF, n), jnp.bfloat16),
            pltpu.SemaphoreType.DMA((2, NSUB, NHOP)),
            pltpu.SemaphoreType.DMA((2, NSUB, NHOP)),
            pltpu.SemaphoreType.DMA((2, N_DEV)),
            pltpu.SemaphoreType.DMA((2 * N_DEV * NSUB,)),
        ],
        compiler_params=pltpu.CompilerParams(collective_id=0),
    )(t)


# device time: 46635 ns/iter; 1.0129x vs baseline; 1.0129x over previous
import jax
import jax.numpy as jnp
from jax import lax
from jax.experimental import pallas as pl
from jax.experimental.pallas import tpu as pltpu

N_DEV = 4
CH = 256
NSUB = 2
SUB = CH // NSUB
HALF = N_DEV * CH
NHOP = 2 * (N_DEV - 1)


def _f(s):
    r = jnp.maximum(s, 0.0)
    return jnp.tanh(s) * s * s + r * r * r


def kernel(t):
    m, n = t.shape
    assert m == 2 * HALF

    def body(x_hbm, out_hbm, x_v, out_v, rbf, sbf, agb,
             ssem, rsem, insem, outsem):
        my = lax.axis_index("i")
        right = lax.rem(my + 1, N_DEV)
        left = lax.rem(my + 3, N_DEV)

        def rows(d, q, s=None):
            if s is None:
                return pl.ds(d * HALF + q * CH, CH)
            return pl.ds(d * HALF + q * CH + s * SUB, SUB)

        def agrows(q, s):
            return pl.ds(q * CH + s * SUB, SUB)

        def chunk_id(d, h):
            if h < N_DEV - 1:
                delta = -h if d == 0 else h
            else:
                ha = h - (N_DEV - 1)
                delta = 1 - ha if d == 0 else -1 + ha
            return lax.rem(my + delta + 2 * N_DEV, N_DEV)

        in_copies = {}
        for k, (df, dr) in enumerate(zip([0, -1, -2, 1], [0, 1, 2, -1])):
            for d, delta in ((0, df), (1, dr)):
                c = lax.rem(my + delta + N_DEV, N_DEV)
                cp = pltpu.make_async_copy(
                    x_hbm.at[rows(d, c)], x_v.at[rows(d, c)], insem.at[d, k]
                )
                cp.start()
                in_copies[(d, k)] = cp

        barrier_sem = pltpu.get_barrier_semaphore()
        for nbr in [left, right]:
            pl.semaphore_signal(
                barrier_sem, inc=1,
                device_id=(nbr,), device_id_type=pl.DeviceIdType.MESH,
            )

        for d in (0, 1):
            in_copies[(d, 0)].wait()
            c0 = chunk_id(d, 0)
            sbf[d, 0, :, :] = x_v[rows(d, c0), :].astype(jnp.bfloat16)

        pl.semaphore_wait(barrier_sem, 2)

        def rdma(src, dst, d, s, h, dev):
            return pltpu.make_async_remote_copy(
                src_ref=src, dst_ref=dst,
                send_sem=ssem.at[d, s, h], recv_sem=rsem.at[d, s, h],
                device_id=(dev,), device_id_type=pl.DeviceIdType.MESH,
            )

        dev_of = {0: right, 1: left}
        started = {}
        out_copies = []
        order = [(d, s) for s in range(NSUB) for d in (0, 1)]

        def copy_out(region):
            cp = pltpu.make_async_copy(
                out_v.at[region], out_hbm.at[region], outsem.at[len(out_copies)]
            )
            cp.start()
            out_copies.append(cp)

        for h in range(N_DEV - 1):
            for d, s in order:
                c = chunk_id(d, h)
                if s == 0 and h > 0:
                    in_copies[(d, h)].wait()
                sub = pl.ds(s * SUB, SUB)
                if h == 0:
                    pass
                else:
                    started[(d, s, h - 1)].wait_recv()
                    sbf[d, h, sub, :] = (
                        rbf[d, h - 1, sub, :].astype(jnp.float32)
                        + x_v[rows(d, c, s), :]
                    ).astype(jnp.bfloat16)
                r = rdma(sbf.at[d, h, sub], rbf.at[d, h, sub], d, s, h, dev_of[d])
                r.start()
                started[(d, s, h)] = r

        for d, s in order:
            started[(d, s, N_DEV - 2)].wait_recv()
            if s == 0:
                in_copies[(d, N_DEV - 1)].wait()
            c = chunk_id(d, N_DEV - 1)
            sub = pl.ds(s * SUB, SUB)
            y = _f(
                rbf[d, N_DEV - 2, sub, :].astype(jnp.float32)
                + x_v[rows(d, c, s), :]
            )
            out_v[rows(d, c, s), :] = y
            agb[d, agrows(c, s), :] = y.astype(jnp.bfloat16)
            r = rdma(agb.at[d, agrows(c, s)], agb.at[d, agrows(c, s)],
                     d, s, N_DEV - 1, dev_of[d])
            r.start()
            started[(d, s, N_DEV - 1)] = r
            copy_out(rows(d, c, s))

        for h in range(N_DEV, NHOP):
            for d, s in order:
                started[(d, s, h - 1)].wait_recv()
                c = chunk_id(d, h)
                r = rdma(agb.at[d, agrows(c, s)], agb.at[d, agrows(c, s)],
                         d, s, h, dev_of[d])
                r.start()
                started[(d, s, h)] = r
                out_v[rows(d, c, s), :] = agb[d, agrows(c, s), :].astype(
                    jnp.float32
                )
                copy_out(rows(d, c, s))

        for d, s in order:
            started[(d, s, NHOP - 1)].wait_recv()
            cf = chunk_id(d, NHOP)
            out_v[rows(d, cf, s), :] = agb[d, agrows(cf, s), :].astype(
                jnp.float32
            )
            copy_out(rows(d, cf, s))
        for cp in out_copies:
            cp.wait()
        for r in started.values():
            r.wait_send()

    return pl.pallas_call(
        body,
        out_shape=jax.ShapeDtypeStruct((m, n), jnp.float32),
        in_specs=[pl.BlockSpec(memory_space=pl.ANY)],
        out_specs=pl.BlockSpec(memory_space=pl.ANY),
        scratch_shapes=[
            pltpu.VMEM((m, n), jnp.float32),
            pltpu.VMEM((m, n), jnp.float32),
            pltpu.VMEM((2, N_DEV - 1, CH, n), jnp.bfloat16),
            pltpu.VMEM((2, N_DEV - 1, CH, n), jnp.bfloat16),
            pltpu.VMEM((2, HALF, n), jnp.bfloat16),
            pltpu.SemaphoreType.DMA((2, NSUB, NHOP)),
            pltpu.SemaphoreType.DMA((2, NSUB, NHOP)),
            pltpu.SemaphoreType.DMA((2, N_DEV)),
            pltpu.SemaphoreType.DMA((2 * N_DEV * NSUB,)),
        ],
        compiler_params=pltpu.CompilerParams(collective_id=0),
    )(t)
